# baseline (speedup 1.0000x reference)
import functools

import jax
import jax.numpy as jnp
from jax import lax
from jax.experimental import pallas as pl
from jax.experimental.pallas import tpu as pltpu
from jax.experimental.pallas import tpu_sc as plsc

NUM_USER = 10000
NUM_ITEM = 10000
R = 5
EMBED = 64
GLEN = 32
D = EMBED + GLEN
AGG = 256
OUT = 128
E = 320000
EDEC = 100000

NC = 2
NS = 16
NW = NC * NS

EPW = E // NW
NB_E = 80
EPT = NB_E * 128
GRP_E = 8

DPW = EDEC // NW
NB_D = 25
DPT = NB_D * 128
GRP_D = 5
EPD = NW * DPT

NP = 10240
NR = R * NP
DUMMY = NUM_USER
STRIPE = NR // NS
UPT = NP // NS

_MESH = plsc.VectorSubcoreMesh(core_axis_name="c", subcore_axis_name="s")
_SC_PARAMS = pltpu.CompilerParams(use_tc_tiling_on_sc=False)


def _leaky(x):
    return jnp.where(x >= 0, x, 0.01 * x)


def _dot(a, b):
    return jax.lax.dot_general(
        a, b, (((a.ndim - 1,), (0,)), ((), ())),
        precision=jax.lax.Precision.HIGHEST,
        preferred_element_type=jnp.float32)


def _dot_fast(a, b):
    return jax.lax.dot_general(
        a, b, (((a.ndim - 1,), (0,)), ((), ())),
        preferred_element_type=jnp.float32)


def _writeback(acc, out_hbm, c, cid, sid, sem):
    wds = [pltpu.async_copy(
        acc.at[pl.ds(t * NP + sid * UPT, UPT)],
        out_hbm.at[c, cid, pl.ds(sid * UPT, UPT), pl.ds(16 * t, 16)], sem)
        for t in range(R)]
    for d in wds:
        d.wait()


def _zero_stripe(zbuf, acc, base, sem):
    zds = [pltpu.async_copy(zbuf, acc.at[pl.ds(base + j * 128, 128)], sem)
           for j in range(STRIPE // 128)]
    for d in zds:
        d.wait()


@functools.partial(
    pl.kernel,
    out_type=jax.ShapeDtypeStruct((2, NC, NP, 128), jnp.float32),
    mesh=_MESH,
    compiler_params=_SC_PARAMS,
    scratch_types=[
        pltpu.VMEM((NB_E, 128), jnp.int32),
        pltpu.VMEM((NB_E, 128), jnp.int32),
        pltpu.VMEM((128, 16), jnp.float32),
        pltpu.VMEM((128, 16), jnp.float32),
        pltpu.VMEM_SHARED((NR, 16), jnp.float32),
        pltpu.SemaphoreType.DMA,
    ],
)
def _deg_kernel(dstu_hbm, dsti_hbm, ones_hbm, zero_hbm, out_hbm,
                idxu, idxi, cbuf, zbuf, acc, sem):
    cid = lax.axis_index("c")
    sid = lax.axis_index("s")
    wid = sid * NC + cid
    base = sid * STRIPE

    pltpu.sync_copy(dstu_hbm.at[wid], idxu)
    pltpu.sync_copy(dsti_hbm.at[wid], idxi)
    pltpu.sync_copy(ones_hbm, cbuf)
    pltpu.sync_copy(zero_hbm, zbuf)

    for which, idx in ((0, idxu), (1, idxi)):
        _zero_stripe(zbuf, acc, base, sem)
        plsc.subcore_barrier()

        def group_body(g, c):
            descs = []
            for k in range(GRP_E):
                b = g * GRP_E + k
                descs.append(pltpu.async_copy(
                    cbuf, acc.at[idx.at[b]], sem, add=True))
            for d in descs:
                d.wait()
            return c
        lax.fori_loop(0, NB_E // GRP_E, group_body, 0)
        plsc.subcore_barrier()

        _writeback(acc, out_hbm, which, cid, sid, sem)
        plsc.subcore_barrier()


def _make_chunk_kernel(kinds):
    n = len(kinds)

    @functools.partial(
        pl.kernel,
        out_type=jax.ShapeDtypeStruct((n, NC, NP, 128), jnp.float32),
        mesh=_MESH,
        compiler_params=_SC_PARAMS,
        scratch_types=(
            [pltpu.VMEM((NB_E, 128), jnp.int32) for _ in range(4)]
            + [
                pltpu.VMEM((1, GRP_E, 128, 16), jnp.float32),
                pltpu.VMEM((128, 16), jnp.float32),
                pltpu.VMEM_SHARED((NR, 16), jnp.float32),
                pltpu.SemaphoreType.DMA,
                pltpu.SemaphoreType.DMA,
            ]
        ),
    )
    def chunk_kernel(*refs):
        tables = refs[:n]
        srcA_hbm, dstA_hbm, srcB_hbm, dstB_hbm, zero_hbm, out_hbm = refs[n:n + 6]
        isA, idA, isB, idB, buf, zbuf, acc, gsem, ssem = refs[n + 6:]

        cid = lax.axis_index("c")
        sid = lax.axis_index("s")
        wid = sid * NC + cid
        base = sid * STRIPE

        if "A" in kinds:
            pltpu.sync_copy(srcA_hbm.at[wid], isA)
            pltpu.sync_copy(dstA_hbm.at[wid], idA)
        if "B" in kinds:
            pltpu.sync_copy(srcB_hbm.at[wid], isB)
            pltpu.sync_copy(dstB_hbm.at[wid], idB)
        pltpu.sync_copy(zero_hbm, zbuf)

        for c, kind in enumerate(kinds):
            src = isA if kind == "A" else isB
            dst = idA if kind == "A" else idB
            tab = tables[c]

            _zero_stripe(zbuf, acc, base, gsem)
            plsc.subcore_barrier()

            def group_body(g, carry):
                gds = []
                for k in range(GRP_E):
                    b = g * GRP_E + k
                    gds.append(pltpu.async_copy(
                        tab.at[src.at[b]], buf.at[0, k], gsem))
                sds = []
                for k in range(GRP_E):
                    b = g * GRP_E + k
                    gds[k].wait()
                    sds.append(pltpu.async_copy(
                        buf.at[0, k], acc.at[dst.at[b]], ssem, add=True))
                for d in sds:
                    d.wait()
                return carry
            lax.fori_loop(0, NB_E // GRP_E, group_body, 0)
            plsc.subcore_barrier()

            _writeback(acc, out_hbm, c, cid, sid, gsem)
            plsc.subcore_barrier()

    return chunk_kernel


_chunkA_kernel = _make_chunk_kernel(("A",) * 7 + ("B",) * 4)
_chunkB_kernel = _make_chunk_kernel(("B", "B"))


@functools.partial(
    pl.kernel,
    out_type=(jax.ShapeDtypeStruct((EPD, OUT), jnp.float32),
              jax.ShapeDtypeStruct((EPD, OUT), jnp.float32)),
    mesh=_MESH,
    compiler_params=_SC_PARAMS,
    scratch_types=[
        pltpu.VMEM((NB_D, 128), jnp.int32),
        pltpu.VMEM((NB_D, 128), jnp.int32),
        pltpu.VMEM((1, GRP_D, 128, OUT), jnp.float32),
        pltpu.SemaphoreType.DMA,
        pltpu.SemaphoreType.DMA,
    ],
)
def _dec_gather_kernel(hu_hbm, hi_hbm, ids_hbm, idd_hbm, g1_hbm, g2_hbm,
                       idxs, idxd, buf, sem1, sem2):
    cid = lax.axis_index("c")
    sid = lax.axis_index("s")
    wid = sid * NC + cid

    pltpu.sync_copy(ids_hbm.at[wid], idxs)
    pltpu.sync_copy(idd_hbm.at[wid], idxd)

    for tab, idx, out in ((hu_hbm, idxs, g1_hbm), (hi_hbm, idxd, g2_hbm)):
        def group_body(g, carry):
            gds = []
            for k in range(GRP_D):
                b = g * GRP_D + k
                gds.append(pltpu.async_copy(tab.at[idx.at[b]],
                                            buf.at[0, k], sem1))
            wds = []
            for k in range(GRP_D):
                b = g * GRP_D + k
                row = wid * DPT + b * 128
                gds[k].wait()
                wds.append(pltpu.async_copy(buf.at[0, k],
                                            out.at[pl.ds(row, 128)], sem2))
            for d in wds:
                d.wait()
            return carry
        lax.fori_loop(0, NB_D // GRP_D, group_body, 0)


_NBLK = 512
_NG = NP // _NBLK
_ZPAD = 128 - 16 * R


def _tc1_body(du_ref, di_ref, emb_ref, ife_ref, ue_ref, *outs):
    emb = emb_ref[...]
    ife = ife_ref[...]
    ue = ue_ref[...]
    z = jnp.zeros((_NBLK, _ZPAD), jnp.float32)
    nus, nis = [], []
    for t in range(R):
        du = (du_ref[0, 0, :, 16 * t:16 * t + 1]
              + du_ref[0, 1, :, 16 * t:16 * t + 1])
        di = (di_ref[0, 0, :, 16 * t:16 * t + 1]
              + di_ref[0, 1, :, 16 * t:16 * t + 1])
        nus.append(lax.rsqrt(jnp.maximum(du, 1.0)))
        nis.append(lax.rsqrt(jnp.maximum(di, 1.0)))
    ocol = jnp.ones((_NBLK, 1), jnp.float32)
    zcol = jnp.zeros((_NBLK, 15), jnp.float32)
    srcs = ([emb[:, 16 * c:16 * c + 16] for c in range(4)]
            + [ife[:, :16], ife[:, 16:],
               jnp.concatenate([ocol, zcol], axis=1)])
    for c in range(7):
        outs[c][...] = jnp.concatenate(
            [srcs[c] * nis[t] for t in range(R)] + [z], axis=1)
    for c in range(4):
        outs[7 + c][...] = jnp.concatenate(
            [ue[:, 16 * c:16 * c + 16] * nus[t] for t in range(R)] + [z],
            axis=1)


def _run_tc1(degp, item_embed, ifeat, user_embed):
    deg_spec = [
        pl.BlockSpec((1, 2, _NBLK, 128), lambda b, w=w: (w, 0, b, 0))
        for w in range(2)
    ]
    node_spec = lambda w: pl.BlockSpec((_NBLK, w), lambda b: (b, 0))
    tab_spec = pl.BlockSpec((_NBLK, 128), lambda b: (b, 0))
    tab_shape = jax.ShapeDtypeStruct((NP, 128), jnp.float32)
    return pl.pallas_call(
        _tc1_body,
        grid=(_NG,),
        in_specs=deg_spec + [node_spec(64), node_spec(32), node_spec(64)],
        out_specs=[tab_spec] * 11,
        out_shape=[tab_shape] * 11,
    )(degp, degp, item_embed, ifeat, user_embed)


def _tc2_body(*refs):
    p_refs = refs[:11]
    du_ref, di_ref = refs[11:13]
    wu_ref, gw_ref, gb_ref, wou_ref, bu_ref, wi_ref = refs[13:19]
    hu2_ref, hipre_ref, tb2a_ref, tb2b_ref = refs[19:]

    wu = wu_ref[...]
    gw = gw_ref[...]
    gb = gb_ref[...]
    wi = wi_ref[...]
    ps = [r[...] for r in p_refs]

    hu_pre = jnp.zeros((_NBLK, AGG), jnp.float32)
    hi_pre = jnp.zeros((_NBLK, AGG), jnp.float32)
    ufeat = jnp.zeros((_NBLK, GLEN), jnp.float32)
    nus = []
    for t in range(R):
        sl = slice(16 * t, 16 * t + 16)
        du = (du_ref[0, 0, :, 16 * t:16 * t + 1]
              + du_ref[0, 1, :, 16 * t:16 * t + 1])
        di = (di_ref[0, 0, :, 16 * t:16 * t + 1]
              + di_ref[0, 1, :, 16 * t:16 * t + 1])
        nu = lax.rsqrt(jnp.maximum(du, 1.0))
        ni = lax.rsqrt(jnp.maximum(di, 1.0))
        nus.append(nu)
        a = jnp.concatenate(
            [ps[c][0, 0, :, sl] + ps[c][0, 1, :, sl] for c in range(6)],
            axis=1) * nu
        s = (ps[6][0, 0, :, 16 * t:16 * t + 1]
             + ps[6][0, 1, :, 16 * t:16 * t + 1]) * nu
        hu_pre = hu_pre + _dot(a, wu[t])
        ufeat = ufeat + _dot(a[:, 64:96], gw[t]) + s * gb[t:t + 1, :]
        b64 = jnp.concatenate(
            [ps[7 + c][0, 0, :, sl] + ps[7 + c][0, 1, :, sl]
             for c in range(4)], axis=1) * ni
        hi_pre = hi_pre + _dot(b64, wi[t, :64, :])

    hu = _leaky(hu_pre)
    hu2_ref[...] = _leaky(_dot(hu, wou_ref[...]) + bu_ref[...])
    hipre_ref[...] = hi_pre
    z = jnp.zeros((_NBLK, _ZPAD), jnp.float32)
    tb2a_ref[...] = jnp.concatenate(
        [ufeat[:, :16] * nus[t] for t in range(R)] + [z], axis=1)
    tb2b_ref[...] = jnp.concatenate(
        [ufeat[:, 16:] * nus[t] for t in range(R)] + [z], axis=1)


def _run_tc2(P, degp, enc_Wu, genre_W, genre_b, enc_out_Wu, bu_row, enc_Wi):
    p_spec = [
        pl.BlockSpec((1, 2, _NBLK, 128), lambda b, c=c: (c, 0, b, 0))
        for c in range(11)
    ]
    deg_spec = [
        pl.BlockSpec((1, 2, _NBLK, 128), lambda b, w=w: (w, 0, b, 0))
        for w in range(2)
    ]
    full = lambda *s: pl.BlockSpec(s, lambda b: tuple(0 for _ in s))
    return pl.pallas_call(
        _tc2_body,
        grid=(_NG,),
        in_specs=(p_spec + deg_spec + [
            full(R, D, AGG), full(R, GLEN, GLEN), full(R, GLEN),
            full(AGG, OUT), full(1, OUT), full(R, D, AGG),
        ]),
        out_specs=[
            pl.BlockSpec((_NBLK, OUT), lambda b: (b, 0)),
            pl.BlockSpec((_NBLK, AGG), lambda b: (b, 0)),
            pl.BlockSpec((_NBLK, 128), lambda b: (b, 0)),
            pl.BlockSpec((_NBLK, 128), lambda b: (b, 0)),
        ],
        out_shape=[
            jax.ShapeDtypeStruct((NP, OUT), jnp.float32),
            jax.ShapeDtypeStruct((NP, AGG), jnp.float32),
            jax.ShapeDtypeStruct((NP, 128), jnp.float32),
            jax.ShapeDtypeStruct((NP, 128), jnp.float32),
        ],
    )(*([P] * 11), degp, degp, enc_Wu, genre_W, genre_b, enc_out_Wu,
      bu_row, enc_Wi)


def _tc3_body(qa_ref, qb_ref, di_ref, hipre_ref, wi_ref, woi_ref, bi_ref,
              hi2_ref):
    wi = wi_ref[...]
    hi_pre = hipre_ref[...]
    for t in range(R):
        sl = slice(16 * t, 16 * t + 16)
        di = (di_ref[0, 0, :, 16 * t:16 * t + 1]
              + di_ref[0, 1, :, 16 * t:16 * t + 1])
        ni = lax.rsqrt(jnp.maximum(di, 1.0))
        c32 = jnp.concatenate(
            [qa_ref[0, 0, :, sl] + qa_ref[0, 1, :, sl],
             qb_ref[0, 0, :, sl] + qb_ref[0, 1, :, sl]], axis=1) * ni
        hi_pre = hi_pre + _dot(c32, wi[t, 64:96, :])
    hi = _leaky(hi_pre)
    hi2_ref[...] = _leaky(_dot(hi, woi_ref[...]) + bi_ref[...])


def _run_tc3(Q, degp, hi_pre1, enc_Wi, enc_out_Wi, bi_row):
    q_spec = [
        pl.BlockSpec((1, 2, _NBLK, 128), lambda b, c=c: (c, 0, b, 0))
        for c in range(2)
    ]
    di_spec = pl.BlockSpec((1, 2, _NBLK, 128), lambda b: (1, 0, b, 0))
    full = lambda *s: pl.BlockSpec(s, lambda b: tuple(0 for _ in s))
    return pl.pallas_call(
        _tc3_body,
        grid=(_NG,),
        in_specs=(q_spec + [
            di_spec,
            pl.BlockSpec((_NBLK, AGG), lambda b: (b, 0)),
            full(R, D, AGG), full(AGG, OUT), full(1, OUT),
        ]),
        out_specs=pl.BlockSpec((_NBLK, OUT), lambda b: (b, 0)),
        out_shape=jax.ShapeDtypeStruct((NP, OUT), jnp.float32),
    )(Q, Q, degp, hi_pre1, enc_Wi, enc_out_Wi, bi_row)


_EBLK = 1024


def _tc4_body(g1_ref, g2_ref, w1_ref, b1_ref, w2_ref, b2_ref, out_ref):
    w1 = w1_ref[...]
    h = _dot_fast(g1_ref[...], w1[:OUT, :]) + _dot_fast(g2_ref[...],
                                                        w1[OUT:, :])
    h = jnp.maximum(h + b1_ref[...], 0.0)
    out_ref[...] = _dot_fast(h, w2_ref[...]) + b2_ref[...]


def _run_tc4(G1, G2, pred_W1, b1_row, pred_W2, b2_arr):
    full = lambda *s: pl.BlockSpec(s, lambda b: tuple(0 for _ in s))
    return pl.pallas_call(
        _tc4_body,
        grid=(EPD // _EBLK,),
        in_specs=[
            pl.BlockSpec((_EBLK, OUT), lambda b: (b, 0)),
            pl.BlockSpec((_EBLK, OUT), lambda b: (b, 0)),
            full(2 * OUT, OUT), full(1, OUT), full(OUT, 1), full(1, 1),
        ],
        out_specs=pl.BlockSpec((_EBLK, 1), lambda b: (b, 0)),
        out_shape=jax.ShapeDtypeStruct((EPD, 1), jnp.float32),
    )(G1, G2, pred_W1, b1_row, pred_W2, b2_arr)


def _pack_edges(idx, pad_val, nb):
    per = idx.shape[0] // NW
    idx = idx.reshape(NW, per)
    pad = jnp.full((NW, nb * 128 - per), pad_val, jnp.int32)
    return jnp.concatenate([idx, pad], axis=1).reshape(NW, nb, 128)


def kernel(ifeat, edge_index, edge_type, dec_edge_index, user_embed,
           item_embed, genre_W, genre_b, enc_Wu, enc_Wi, enc_out_Wu,
           enc_out_bu, enc_out_Wi, enc_out_bi, pred_W1, pred_b1, pred_W2,
           pred_b2):
    u = edge_index[0].astype(jnp.int32)
    i = edge_index[1].astype(jnp.int32)
    t = edge_type.astype(jnp.int32)

    dst_u = _pack_edges(t * NP + u, DUMMY, NB_E)
    dst_i = _pack_edges(t * NP + i, DUMMY, NB_E)
    src_i = _pack_edges(i * 8 + t, 0, NB_E)
    src_u = _pack_edges(u * 8 + t, 0, NB_E)
    dec_s = _pack_edges(dec_edge_index[0].astype(jnp.int32), 0, NB_D)
    dec_d = _pack_edges(dec_edge_index[1].astype(jnp.int32), 0, NB_D)

    ones16 = jnp.zeros((128, 16), jnp.float32).at[:, 0].set(1.0)
    zeros16 = jnp.zeros((128, 16), jnp.float32)

    degp = _deg_kernel(dst_u, dst_i, ones16, zeros16)

    tabs = [x.reshape(NP * 8, 16)
            for x in _run_tc1(degp, item_embed, ifeat, user_embed)]

    P = _chunkA_kernel(*tabs, src_i, dst_u, src_u, dst_i, zeros16)

    hu2, hi_pre1, tb2a, tb2b = _run_tc2(
        P, degp, enc_Wu, genre_W, genre_b, enc_out_Wu,
        enc_out_bu.reshape(1, OUT), enc_Wi)

    Q = _chunkB_kernel(tb2a.reshape(NP * 8, 16), tb2b.reshape(NP * 8, 16),
                       src_i, dst_u, src_u, dst_i, zeros16)

    hi2 = _run_tc3(Q, degp, hi_pre1, enc_Wi, enc_out_Wi,
                   enc_out_bi.reshape(1, OUT))

    G1, G2 = _dec_gather_kernel(hu2, hi2, dec_s, dec_d)

    sc = _run_tc4(G1, G2, pred_W1, pred_b1.reshape(1, OUT), pred_W2,
                  pred_b2.reshape(1, 1))

    return sc.reshape(NW, DPT)[:, :DPW].reshape(EDEC)


# speedup vs baseline: 1.0404x; 1.0404x over previous
import functools

import jax
import jax.numpy as jnp
from jax import lax
from jax.experimental import pallas as pl
from jax.experimental.pallas import tpu as pltpu
from jax.experimental.pallas import tpu_sc as plsc

NUM_USER = 10000
NUM_ITEM = 10000
R = 5
EMBED = 64
GLEN = 32
D = EMBED + GLEN
AGG = 256
OUT = 128
E = 320000
EDEC = 100000

NC = 2
NS = 16
NW = NC * NS

EPW = E // NW
NB_E = 80
EPT = NB_E * 128
GRP_E = 16

DPW = EDEC // NW
NB_D = 25
DPT = NB_D * 128
GRP_D = 5
EPD = NW * DPT

NP = 10240
NR = R * NP
DUMMY = NUM_USER
STRIPE = NR // NS
UPT = NP // NS

_MESH = plsc.VectorSubcoreMesh(core_axis_name="c", subcore_axis_name="s")
_SC_PARAMS = pltpu.CompilerParams(use_tc_tiling_on_sc=False)


def _leaky(x):
    return jnp.where(x >= 0, x, 0.01 * x)


def _dot(a, b):
    return jax.lax.dot_general(
        a, b, (((a.ndim - 1,), (0,)), ((), ())),
        precision=jax.lax.Precision.HIGHEST,
        preferred_element_type=jnp.float32)


def _dot_fast(a, b):
    return jax.lax.dot_general(
        a, b, (((a.ndim - 1,), (0,)), ((), ())),
        preferred_element_type=jnp.float32)


def _writeback(acc, out_hbm, c, cid, sid, sem):
    wds = [pltpu.async_copy(
        acc.at[pl.ds(t * NP + sid * UPT, UPT)],
        out_hbm.at[c, cid, pl.ds(sid * UPT, UPT), pl.ds(16 * t, 16)], sem)
        for t in range(R)]
    for d in wds:
        d.wait()


def _zero_stripe(zbuf, acc, base, sem):
    zds = [pltpu.async_copy(zbuf, acc.at[pl.ds(base + j * 128, 128)], sem)
           for j in range(STRIPE // 128)]
    for d in zds:
        d.wait()


@functools.partial(
    pl.kernel,
    out_type=jax.ShapeDtypeStruct((2, NC, NP, 128), jnp.float32),
    mesh=_MESH,
    compiler_params=_SC_PARAMS,
    scratch_types=[
        pltpu.VMEM((NB_E, 128), jnp.int32),
        pltpu.VMEM((NB_E, 128), jnp.int32),
        pltpu.VMEM((128, 16), jnp.float32),
        pltpu.VMEM((128, 16), jnp.float32),
        pltpu.VMEM_SHARED((NR, 16), jnp.float32),
        pltpu.SemaphoreType.DMA,
    ],
)
def _deg_kernel(dstu_hbm, dsti_hbm, ones_hbm, zero_hbm, out_hbm,
                idxu, idxi, cbuf, zbuf, acc, sem):
    cid = lax.axis_index("c")
    sid = lax.axis_index("s")
    wid = sid * NC + cid
    base = sid * STRIPE

    pltpu.sync_copy(dstu_hbm.at[wid], idxu)
    pltpu.sync_copy(dsti_hbm.at[wid], idxi)
    pltpu.sync_copy(ones_hbm, cbuf)
    pltpu.sync_copy(zero_hbm, zbuf)

    for which, idx in ((0, idxu), (1, idxi)):
        _zero_stripe(zbuf, acc, base, sem)
        plsc.subcore_barrier()

        def group_body(g, c):
            descs = []
            for k in range(GRP_E):
                b = g * GRP_E + k
                descs.append(pltpu.async_copy(
                    cbuf, acc.at[idx.at[b]], sem, add=True))
            for d in descs:
                d.wait()
            return c
        lax.fori_loop(0, NB_E // GRP_E, group_body, 0)
        plsc.subcore_barrier()

        _writeback(acc, out_hbm, which, cid, sid, sem)
        plsc.subcore_barrier()


def _make_chunk_kernel(kinds):
    n = len(kinds)

    @functools.partial(
        pl.kernel,
        out_type=jax.ShapeDtypeStruct((n, NC, NP, 128), jnp.float32),
        mesh=_MESH,
        compiler_params=_SC_PARAMS,
        scratch_types=(
            [pltpu.VMEM((NB_E, 128), jnp.int32) for _ in range(4)]
            + [
                pltpu.VMEM((1, GRP_E, 128, 16), jnp.float32),
                pltpu.VMEM((128, 16), jnp.float32),
                pltpu.VMEM_SHARED((NR, 16), jnp.float32),
                pltpu.SemaphoreType.DMA,
                pltpu.SemaphoreType.DMA,
            ]
        ),
    )
    def chunk_kernel(*refs):
        tables = refs[:n]
        srcA_hbm, dstA_hbm, srcB_hbm, dstB_hbm, zero_hbm, out_hbm = refs[n:n + 6]
        isA, idA, isB, idB, buf, zbuf, acc, gsem, ssem = refs[n + 6:]

        cid = lax.axis_index("c")
        sid = lax.axis_index("s")
        wid = sid * NC + cid
        base = sid * STRIPE

        if "A" in kinds:
            pltpu.sync_copy(srcA_hbm.at[wid], isA)
            pltpu.sync_copy(dstA_hbm.at[wid], idA)
        if "B" in kinds:
            pltpu.sync_copy(srcB_hbm.at[wid], isB)
            pltpu.sync_copy(dstB_hbm.at[wid], idB)
        pltpu.sync_copy(zero_hbm, zbuf)

        for c, kind in enumerate(kinds):
            src = isA if kind == "A" else isB
            dst = idA if kind == "A" else idB
            tab = tables[c]

            _zero_stripe(zbuf, acc, base, gsem)
            plsc.subcore_barrier()

            def group_body(g, carry):
                gds = []
                for k in range(GRP_E):
                    b = g * GRP_E + k
                    gds.append(pltpu.async_copy(
                        tab.at[src.at[b]], buf.at[0, k], gsem))
                sds = []
                for k in range(GRP_E):
                    b = g * GRP_E + k
                    gds[k].wait()
                    sds.append(pltpu.async_copy(
                        buf.at[0, k], acc.at[dst.at[b]], ssem, add=True))
                for d in sds:
                    d.wait()
                return carry
            lax.fori_loop(0, NB_E // GRP_E, group_body, 0)
            plsc.subcore_barrier()

            _writeback(acc, out_hbm, c, cid, sid, gsem)
            plsc.subcore_barrier()

    return chunk_kernel


_chunkA_kernel = _make_chunk_kernel(("A",) * 7 + ("B",) * 4)
_chunkB_kernel = _make_chunk_kernel(("B", "B"))


@functools.partial(
    pl.kernel,
    out_type=(jax.ShapeDtypeStruct((EPD, OUT), jnp.float32),
              jax.ShapeDtypeStruct((EPD, OUT), jnp.float32)),
    mesh=_MESH,
    compiler_params=_SC_PARAMS,
    scratch_types=[
        pltpu.VMEM((NB_D, 128), jnp.int32),
        pltpu.VMEM((NB_D, 128), jnp.int32),
        pltpu.VMEM((1, GRP_D, 128, OUT), jnp.float32),
        pltpu.SemaphoreType.DMA,
        pltpu.SemaphoreType.DMA,
    ],
)
def _dec_gather_kernel(hu_hbm, hi_hbm, ids_hbm, idd_hbm, g1_hbm, g2_hbm,
                       idxs, idxd, buf, sem1, sem2):
    cid = lax.axis_index("c")
    sid = lax.axis_index("s")
    wid = sid * NC + cid

    pltpu.sync_copy(ids_hbm.at[wid], idxs)
    pltpu.sync_copy(idd_hbm.at[wid], idxd)

    for tab, idx, out in ((hu_hbm, idxs, g1_hbm), (hi_hbm, idxd, g2_hbm)):
        def group_body(g, carry):
            gds = []
            for k in range(GRP_D):
                b = g * GRP_D + k
                gds.append(pltpu.async_copy(tab.at[idx.at[b]],
                                            buf.at[0, k], sem1))
            wds = []
            for k in range(GRP_D):
                b = g * GRP_D + k
                row = wid * DPT + b * 128
                gds[k].wait()
                wds.append(pltpu.async_copy(buf.at[0, k],
                                            out.at[pl.ds(row, 128)], sem2))
            for d in wds:
                d.wait()
            return carry
        lax.fori_loop(0, NB_D // GRP_D, group_body, 0)


_NBLK = 512
_NG = NP // _NBLK
_ZPAD = 128 - 16 * R


def _tc1_body(du_ref, di_ref, emb_ref, ife_ref, ue_ref, *outs):
    emb = emb_ref[...]
    ife = ife_ref[...]
    ue = ue_ref[...]
    z = jnp.zeros((_NBLK, _ZPAD), jnp.float32)
    nus, nis = [], []
    for t in range(R):
        du = (du_ref[0, 0, :, 16 * t:16 * t + 1]
              + du_ref[0, 1, :, 16 * t:16 * t + 1])
        di = (di_ref[0, 0, :, 16 * t:16 * t + 1]
              + di_ref[0, 1, :, 16 * t:16 * t + 1])
        nus.append(lax.rsqrt(jnp.maximum(du, 1.0)))
        nis.append(lax.rsqrt(jnp.maximum(di, 1.0)))
    ocol = jnp.ones((_NBLK, 1), jnp.float32)
    zcol = jnp.zeros((_NBLK, 15), jnp.float32)
    srcs = ([emb[:, 16 * c:16 * c + 16] for c in range(4)]
            + [ife[:, :16], ife[:, 16:],
               jnp.concatenate([ocol, zcol], axis=1)])
    for c in range(7):
        outs[c][...] = jnp.concatenate(
            [srcs[c] * nis[t] for t in range(R)] + [z], axis=1)
    for c in range(4):
        outs[7 + c][...] = jnp.concatenate(
            [ue[:, 16 * c:16 * c + 16] * nus[t] for t in range(R)] + [z],
            axis=1)


def _run_tc1(degp, item_embed, ifeat, user_embed):
    deg_spec = [
        pl.BlockSpec((1, 2, _NBLK, 128), lambda b, w=w: (w, 0, b, 0))
        for w in range(2)
    ]
    node_spec = lambda w: pl.BlockSpec((_NBLK, w), lambda b: (b, 0))
    tab_spec = pl.BlockSpec((_NBLK, 128), lambda b: (b, 0))
    tab_shape = jax.ShapeDtypeStruct((NP, 128), jnp.float32)
    return pl.pallas_call(
        _tc1_body,
        grid=(_NG,),
        in_specs=deg_spec + [node_spec(64), node_spec(32), node_spec(64)],
        out_specs=[tab_spec] * 11,
        out_shape=[tab_shape] * 11,
    )(degp, degp, item_embed, ifeat, user_embed)


def _tc2_body(*refs):
    p_refs = refs[:11]
    du_ref, di_ref = refs[11:13]
    wu_ref, gw_ref, gb_ref, wou_ref, bu_ref, wi_ref = refs[13:19]
    hu2_ref, hipre_ref, tb2a_ref, tb2b_ref = refs[19:]

    wu = wu_ref[...]
    gw = gw_ref[...]
    gb = gb_ref[...]
    wi = wi_ref[...]
    ps = [r[...] for r in p_refs]

    hu_pre = jnp.zeros((_NBLK, AGG), jnp.float32)
    hi_pre = jnp.zeros((_NBLK, AGG), jnp.float32)
    ufeat = jnp.zeros((_NBLK, GLEN), jnp.float32)
    nus = []
    for t in range(R):
        sl = slice(16 * t, 16 * t + 16)
        du = (du_ref[0, 0, :, 16 * t:16 * t + 1]
              + du_ref[0, 1, :, 16 * t:16 * t + 1])
        di = (di_ref[0, 0, :, 16 * t:16 * t + 1]
              + di_ref[0, 1, :, 16 * t:16 * t + 1])
        nu = lax.rsqrt(jnp.maximum(du, 1.0))
        ni = lax.rsqrt(jnp.maximum(di, 1.0))
        nus.append(nu)
        a = jnp.concatenate(
            [ps[c][0, 0, :, sl] + ps[c][0, 1, :, sl] for c in range(6)],
            axis=1) * nu
        s = (ps[6][0, 0, :, 16 * t:16 * t + 1]
             + ps[6][0, 1, :, 16 * t:16 * t + 1]) * nu
        hu_pre = hu_pre + _dot(a, wu[t])
        ufeat = ufeat + _dot(a[:, 64:96], gw[t]) + s * gb[t:t + 1, :]
        b64 = jnp.concatenate(
            [ps[7 + c][0, 0, :, sl] + ps[7 + c][0, 1, :, sl]
             for c in range(4)], axis=1) * ni
        hi_pre = hi_pre + _dot(b64, wi[t, :64, :])

    hu = _leaky(hu_pre)
    hu2_ref[...] = _leaky(_dot(hu, wou_ref[...]) + bu_ref[...])
    hipre_ref[...] = hi_pre
    z = jnp.zeros((_NBLK, _ZPAD), jnp.float32)
    tb2a_ref[...] = jnp.concatenate(
        [ufeat[:, :16] * nus[t] for t in range(R)] + [z], axis=1)
    tb2b_ref[...] = jnp.concatenate(
        [ufeat[:, 16:] * nus[t] for t in range(R)] + [z], axis=1)


def _run_tc2(P, degp, enc_Wu, genre_W, genre_b, enc_out_Wu, bu_row, enc_Wi):
    p_spec = [
        pl.BlockSpec((1, 2, _NBLK, 128), lambda b, c=c: (c, 0, b, 0))
        for c in range(11)
    ]
    deg_spec = [
        pl.BlockSpec((1, 2, _NBLK, 128), lambda b, w=w: (w, 0, b, 0))
        for w in range(2)
    ]
    full = lambda *s: pl.BlockSpec(s, lambda b: tuple(0 for _ in s))
    return pl.pallas_call(
        _tc2_body,
        grid=(_NG,),
        in_specs=(p_spec + deg_spec + [
            full(R, D, AGG), full(R, GLEN, GLEN), full(R, GLEN),
            full(AGG, OUT), full(1, OUT), full(R, D, AGG),
        ]),
        out_specs=[
            pl.BlockSpec((_NBLK, OUT), lambda b: (b, 0)),
            pl.BlockSpec((_NBLK, AGG), lambda b: (b, 0)),
            pl.BlockSpec((_NBLK, 128), lambda b: (b, 0)),
            pl.BlockSpec((_NBLK, 128), lambda b: (b, 0)),
        ],
        out_shape=[
            jax.ShapeDtypeStruct((NP, OUT), jnp.float32),
            jax.ShapeDtypeStruct((NP, AGG), jnp.float32),
            jax.ShapeDtypeStruct((NP, 128), jnp.float32),
            jax.ShapeDtypeStruct((NP, 128), jnp.float32),
        ],
    )(*([P] * 11), degp, degp, enc_Wu, genre_W, genre_b, enc_out_Wu,
      bu_row, enc_Wi)


def _tc3_body(qa_ref, qb_ref, di_ref, hipre_ref, wi_ref, woi_ref, bi_ref,
              hi2_ref):
    wi = wi_ref[...]
    hi_pre = hipre_ref[...]
    for t in range(R):
        sl = slice(16 * t, 16 * t + 16)
        di = (di_ref[0, 0, :, 16 * t:16 * t + 1]
              + di_ref[0, 1, :, 16 * t:16 * t + 1])
        ni = lax.rsqrt(jnp.maximum(di, 1.0))
        c32 = jnp.concatenate(
            [qa_ref[0, 0, :, sl] + qa_ref[0, 1, :, sl],
             qb_ref[0, 0, :, sl] + qb_ref[0, 1, :, sl]], axis=1) * ni
        hi_pre = hi_pre + _dot(c32, wi[t, 64:96, :])
    hi = _leaky(hi_pre)
    hi2_ref[...] = _leaky(_dot(hi, woi_ref[...]) + bi_ref[...])


def _run_tc3(Q, degp, hi_pre1, enc_Wi, enc_out_Wi, bi_row):
    q_spec = [
        pl.BlockSpec((1, 2, _NBLK, 128), lambda b, c=c: (c, 0, b, 0))
        for c in range(2)
    ]
    di_spec = pl.BlockSpec((1, 2, _NBLK, 128), lambda b: (1, 0, b, 0))
    full = lambda *s: pl.BlockSpec(s, lambda b: tuple(0 for _ in s))
    return pl.pallas_call(
        _tc3_body,
        grid=(_NG,),
        in_specs=(q_spec + [
            di_spec,
            pl.BlockSpec((_NBLK, AGG), lambda b: (b, 0)),
            full(R, D, AGG), full(AGG, OUT), full(1, OUT),
        ]),
        out_specs=pl.BlockSpec((_NBLK, OUT), lambda b: (b, 0)),
        out_shape=jax.ShapeDtypeStruct((NP, OUT), jnp.float32),
    )(Q, Q, degp, hi_pre1, enc_Wi, enc_out_Wi, bi_row)


_EBLK = 1024


def _tc4_body(g1_ref, g2_ref, w1_ref, b1_ref, w2_ref, b2_ref, out_ref):
    w1 = w1_ref[...]
    h = _dot_fast(g1_ref[...], w1[:OUT, :]) + _dot_fast(g2_ref[...],
                                                        w1[OUT:, :])
    h = jnp.maximum(h + b1_ref[...], 0.0)
    out_ref[...] = _dot_fast(h, w2_ref[...]) + b2_ref[...]


def _run_tc4(G1, G2, pred_W1, b1_row, pred_W2, b2_arr):
    full = lambda *s: pl.BlockSpec(s, lambda b: tuple(0 for _ in s))
    return pl.pallas_call(
        _tc4_body,
        grid=(EPD // _EBLK,),
        in_specs=[
            pl.BlockSpec((_EBLK, OUT), lambda b: (b, 0)),
            pl.BlockSpec((_EBLK, OUT), lambda b: (b, 0)),
            full(2 * OUT, OUT), full(1, OUT), full(OUT, 1), full(1, 1),
        ],
        out_specs=pl.BlockSpec((_EBLK, 1), lambda b: (b, 0)),
        out_shape=jax.ShapeDtypeStruct((EPD, 1), jnp.float32),
    )(G1, G2, pred_W1, b1_row, pred_W2, b2_arr)


def _pack_edges(idx, pad_val, nb):
    per = idx.shape[0] // NW
    idx = idx.reshape(NW, per)
    pad = jnp.full((NW, nb * 128 - per), pad_val, jnp.int32)
    return jnp.concatenate([idx, pad], axis=1).reshape(NW, nb, 128)


def kernel(ifeat, edge_index, edge_type, dec_edge_index, user_embed,
           item_embed, genre_W, genre_b, enc_Wu, enc_Wi, enc_out_Wu,
           enc_out_bu, enc_out_Wi, enc_out_bi, pred_W1, pred_b1, pred_W2,
           pred_b2):
    u = edge_index[0].astype(jnp.int32)
    i = edge_index[1].astype(jnp.int32)
    t = edge_type.astype(jnp.int32)

    dst_u = _pack_edges(t * NP + u, DUMMY, NB_E)
    dst_i = _pack_edges(t * NP + i, DUMMY, NB_E)
    src_i = _pack_edges(i * 8 + t, 0, NB_E)
    src_u = _pack_edges(u * 8 + t, 0, NB_E)
    dec_s = _pack_edges(dec_edge_index[0].astype(jnp.int32), 0, NB_D)
    dec_d = _pack_edges(dec_edge_index[1].astype(jnp.int32), 0, NB_D)

    ones16 = jnp.zeros((128, 16), jnp.float32).at[:, 0].set(1.0)
    zeros16 = jnp.zeros((128, 16), jnp.float32)

    degp = _deg_kernel(dst_u, dst_i, ones16, zeros16)

    tabs = [x.reshape(NP * 8, 16)
            for x in _run_tc1(degp, item_embed, ifeat, user_embed)]

    P = _chunkA_kernel(*tabs, src_i, dst_u, src_u, dst_i, zeros16)

    hu2, hi_pre1, tb2a, tb2b = _run_tc2(
        P, degp, enc_Wu, genre_W, genre_b, enc_out_Wu,
        enc_out_bu.reshape(1, OUT), enc_Wi)

    Q = _chunkB_kernel(tb2a.reshape(NP * 8, 16), tb2b.reshape(NP * 8, 16),
                       src_i, dst_u, src_u, dst_i, zeros16)

    hi2 = _run_tc3(Q, degp, hi_pre1, enc_Wi, enc_out_Wi,
                   enc_out_bi.reshape(1, OUT))

    G1, G2 = _dec_gather_kernel(hu2, hi2, dec_s, dec_d)

    sc = _run_tc4(G1, G2, pred_W1, pred_b1.reshape(1, OUT), pred_W2,
                  pred_b2.reshape(1, 1))

    return sc.reshape(NW, DPT)[:, :DPW].reshape(EDEC)
